# baseline (device time: 172317 ns/iter reference)
import jax
import jax.numpy as jnp
from jax import lax
from jax.experimental import pallas as pl
from jax.experimental.pallas import tpu as pltpu

N_Z = 4


def kernel(O, Wo):
    B, S, Hs, D = O.shape
    K = Hs * D
    N = Wo.shape[1]
    s_per = S // N_Z

    x = O.reshape(B, S, K).astype(jnp.bfloat16)
    w = Wo.astype(jnp.bfloat16)
    partial = lax.dot_general(
        x, w, (((2,), (0,)), ((), ())),
        preferred_element_type=jnp.float32,
    ).astype(jnp.bfloat16)

    def body(p_ref, out_ref, comm_ref, send_sems, recv_sems):
        my_x = lax.axis_index("x")
        my_y = lax.axis_index("y")
        my_z = lax.axis_index("z")
        left = (my_z - 1) % N_Z
        right = (my_z + 1) % N_Z

        barrier_sem = pltpu.get_barrier_semaphore()
        for nbr in (left, right):
            pl.semaphore_signal(
                barrier_sem, inc=1,
                device_id=(my_x, my_y, nbr),
                device_id_type=pl.DeviceIdType.MESH,
            )
        pl.semaphore_wait(barrier_sem, 2)

        c0 = (my_z - 1) % N_Z
        comm_ref[0, :, :, :] = p_ref[:, pl.ds(c0 * s_per, s_per), :]

        for h in range(N_Z - 1):
            rdma = pltpu.make_async_remote_copy(
                src_ref=comm_ref.at[h],
                dst_ref=comm_ref.at[h + 1],
                send_sem=send_sems.at[h],
                recv_sem=recv_sems.at[h],
                device_id=(my_x, my_y, right),
                device_id_type=pl.DeviceIdType.MESH,
            )
            rdma.start()
            rdma.wait()

            c = (my_z - 2 - h) % N_Z
            local = p_ref[:, pl.ds(c * s_per, s_per), :]
            if h < N_Z - 2:
                comm_ref[h + 1, :, :, :] = comm_ref[h + 1, :, :, :] + local
            else:
                out_ref[:, :, :] = (
                    comm_ref[h + 1, :, :, :].astype(jnp.float32)
                    + local.astype(jnp.float32)
                )

    return pl.pallas_call(
        body,
        out_shape=jax.ShapeDtypeStruct((B, s_per, N), jnp.float32),
        in_specs=[pl.BlockSpec(memory_space=pltpu.VMEM)],
        out_specs=pl.BlockSpec(memory_space=pltpu.VMEM),
        scratch_shapes=[
            pltpu.VMEM((N_Z, B, s_per, N), jnp.bfloat16),
            pltpu.SemaphoreType.DMA((N_Z - 1,)),
            pltpu.SemaphoreType.DMA((N_Z - 1,)),
        ],
        compiler_params=pltpu.CompilerParams(collective_id=0),
    )(partial)


# device time: 160668 ns/iter; 1.0725x vs baseline; 1.0725x over previous
import jax
import jax.numpy as jnp
from jax import lax
from jax.experimental import pallas as pl
from jax.experimental.pallas import tpu as pltpu

N_Z = 4


def kernel(O, Wo):
    B, S, Hs, D = O.shape
    K = Hs * D
    N = Wo.shape[1]
    s_per = S // N_Z

    x = O.reshape(B, S, K).astype(jnp.bfloat16)
    w = Wo.astype(jnp.bfloat16)

    def body(x_ref, w_ref, out_ref, comm_ref, pbuf_ref, send_sems, recv_sems):
        my_x = lax.axis_index("x")
        my_y = lax.axis_index("y")
        my_z = lax.axis_index("z")
        left = (my_z - 1) % N_Z
        right = (my_z + 1) % N_Z

        def chunk_f32(c, b):
            return lax.dot_general(
                x_ref[b, pl.ds(c * s_per, s_per), :],
                w_ref[:, :],
                (((1,), (0,)), ((), ())),
                preferred_element_type=jnp.float32,
            )

        c0 = (my_z - 1) % N_Z
        for b in range(B):
            comm_ref[0, b, :, :] = chunk_f32(c0, b).astype(jnp.bfloat16)

        barrier_sem = pltpu.get_barrier_semaphore()
        for nbr in (left, right):
            pl.semaphore_signal(
                barrier_sem, inc=1,
                device_id=(my_x, my_y, nbr),
                device_id_type=pl.DeviceIdType.MESH,
            )
        pl.semaphore_wait(barrier_sem, 2)

        rdmas = [
            pltpu.make_async_remote_copy(
                src_ref=comm_ref.at[h],
                dst_ref=comm_ref.at[h + 1],
                send_sem=send_sems.at[h],
                recv_sem=recv_sems.at[h],
                device_id=(my_x, my_y, right),
                device_id_type=pl.DeviceIdType.MESH,
            )
            for h in range(N_Z - 1)
        ]
        rdmas[0].start()

        for h in range(N_Z - 2):
            c = (my_z - 2 - h) % N_Z
            for b in range(B):
                pbuf_ref[h, b, :, :] = chunk_f32(c, b).astype(jnp.bfloat16)
        for b in range(B):
            out_ref[b, :, :] = chunk_f32(my_z, b)

        for h in range(N_Z - 1):
            rdmas[h].wait()
            if h < N_Z - 2:
                comm_ref[h + 1, :, :, :] = (
                    comm_ref[h + 1, :, :, :] + pbuf_ref[h, :, :, :]
                )
                rdmas[h + 1].start()
            else:
                out_ref[:, :, :] = (
                    out_ref[:, :, :]
                    + comm_ref[h + 1, :, :, :].astype(jnp.float32)
                )

    return pl.pallas_call(
        body,
        out_shape=jax.ShapeDtypeStruct((B, s_per, N), jnp.float32),
        in_specs=[
            pl.BlockSpec(memory_space=pltpu.VMEM),
            pl.BlockSpec(memory_space=pltpu.VMEM),
        ],
        out_specs=pl.BlockSpec(memory_space=pltpu.VMEM),
        scratch_shapes=[
            pltpu.VMEM((N_Z, B, s_per, N), jnp.bfloat16),
            pltpu.VMEM((N_Z - 2, B, s_per, N), jnp.bfloat16),
            pltpu.SemaphoreType.DMA((N_Z - 1,)),
            pltpu.SemaphoreType.DMA((N_Z - 1,)),
        ],
        compiler_params=pltpu.CompilerParams(collective_id=0),
    )(x, w)


# device time: 154030 ns/iter; 1.1187x vs baseline; 1.0431x over previous
import jax
import jax.numpy as jnp
from jax import lax
from jax.experimental import pallas as pl
from jax.experimental.pallas import tpu as pltpu

N_Z = 4


def kernel(O, Wo):
    B, S, Hs, D = O.shape
    K = Hs * D
    N = Wo.shape[1]
    s_per = S // N_Z
    n_hops = N_Z - 1

    x = O.reshape(B, S, K).astype(jnp.bfloat16)
    w = Wo.astype(jnp.bfloat16)

    def body(x_ref, w_ref, out_ref, comm_ref, pbuf_ref, send_sems, recv_sems):
        my_x = lax.axis_index("x")
        my_y = lax.axis_index("y")
        my_z = lax.axis_index("z")
        left = (my_z - 1) % N_Z
        right = (my_z + 1) % N_Z

        def chunk_f32(c, b):
            return lax.dot_general(
                x_ref[b, pl.ds(c * s_per, s_per), :],
                w_ref[:, :],
                (((1,), (0,)), ((), ())),
                preferred_element_type=jnp.float32,
            )

        barrier_sem = pltpu.get_barrier_semaphore()
        for nbr in (left, right):
            pl.semaphore_signal(
                barrier_sem, inc=1,
                device_id=(my_x, my_y, nbr),
                device_id_type=pl.DeviceIdType.MESH,
            )
        pl.semaphore_wait(barrier_sem, 2)

        rdmas = [
            [
                pltpu.make_async_remote_copy(
                    src_ref=comm_ref.at[h, b],
                    dst_ref=comm_ref.at[h + 1, b],
                    send_sem=send_sems.at[h, b],
                    recv_sem=recv_sems.at[h, b],
                    device_id=(my_x, my_y, right),
                    device_id_type=pl.DeviceIdType.MESH,
                )
                for b in range(B)
            ]
            for h in range(n_hops)
        ]

        c0 = (my_z - 1) % N_Z
        for b in range(B):
            comm_ref[0, b, :, :] = chunk_f32(c0, b).astype(jnp.bfloat16)
            rdmas[0][b].start()

        for h in range(n_hops - 1):
            c = (my_z - 2 - h) % N_Z
            for b in range(B):
                pbuf_ref[h, b, :, :] = chunk_f32(c, b).astype(jnp.bfloat16)
        for b in range(B):
            out_ref[b, :, :] = chunk_f32(my_z, b)

        for h in range(n_hops):
            for b in range(B):
                rdmas[h][b].wait()
                if h < n_hops - 1:
                    comm_ref[h + 1, b, :, :] = (
                        comm_ref[h + 1, b, :, :] + pbuf_ref[h, b, :, :]
                    )
                    rdmas[h + 1][b].start()
                else:
                    out_ref[b, :, :] = (
                        out_ref[b, :, :]
                        + comm_ref[h + 1, b, :, :].astype(jnp.float32)
                    )

    return pl.pallas_call(
        body,
        out_shape=jax.ShapeDtypeStruct((B, s_per, N), jnp.float32),
        in_specs=[
            pl.BlockSpec(memory_space=pltpu.VMEM),
            pl.BlockSpec(memory_space=pltpu.VMEM),
        ],
        out_specs=pl.BlockSpec(memory_space=pltpu.VMEM),
        scratch_shapes=[
            pltpu.VMEM((N_Z, B, s_per, N), jnp.bfloat16),
            pltpu.VMEM((N_Z - 2, B, s_per, N), jnp.bfloat16),
            pltpu.SemaphoreType.DMA((N_Z - 1, B)),
            pltpu.SemaphoreType.DMA((N_Z - 1, B)),
        ],
        compiler_params=pltpu.CompilerParams(collective_id=0),
    )(x, w)


# device time: 152255 ns/iter; 1.1318x vs baseline; 1.0117x over previous
import jax
import jax.numpy as jnp
from jax import lax
from jax.experimental import pallas as pl
from jax.experimental.pallas import tpu as pltpu

N_Z = 4


def kernel(O, Wo):
    B, S, Hs, D = O.shape
    K = Hs * D
    N = Wo.shape[1]
    s_per = S // N_Z
    n_hops = N_Z - 1

    x = O.reshape(B, S, K)

    def body(x_ref, w_ref, out_ref, comm_ref, wb_ref, send_sems, recv_sems):
        my_x = lax.axis_index("x")
        my_y = lax.axis_index("y")
        my_z = lax.axis_index("z")
        left = (my_z - 1) % N_Z
        right = (my_z + 1) % N_Z

        barrier_sem = pltpu.get_barrier_semaphore()
        for nbr in (left, right):
            pl.semaphore_signal(
                barrier_sem, inc=1,
                device_id=(my_x, my_y, nbr),
                device_id_type=pl.DeviceIdType.MESH,
            )
        pl.semaphore_wait(barrier_sem, 2)

        wb_ref[:, :] = w_ref[:, :].astype(jnp.bfloat16)

        def chunk_f32(c, b):
            xs = x_ref[b, pl.ds(c * s_per, s_per), :].astype(jnp.bfloat16)
            return lax.dot_general(
                xs, wb_ref[:, :],
                (((1,), (0,)), ((), ())),
                preferred_element_type=jnp.float32,
            )

        rdmas = [
            [
                pltpu.make_async_remote_copy(
                    src_ref=comm_ref.at[h, b],
                    dst_ref=comm_ref.at[h + 1, b],
                    send_sem=send_sems.at[h, b],
                    recv_sem=recv_sems.at[h, b],
                    device_id=(my_x, my_y, right),
                    device_id_type=pl.DeviceIdType.MESH,
                )
                for b in range(B)
            ]
            for h in range(n_hops)
        ]

        c0 = (my_z - 1) % N_Z
        for b in range(B):
            comm_ref[0, b, :, :] = chunk_f32(c0, b).astype(jnp.bfloat16)
            rdmas[0][b].start()

        for h in range(n_hops):
            c = (my_z - 2 - h) % N_Z
            for b in range(B):
                if h < n_hops - 1:
                    t = chunk_f32(c, b).astype(jnp.bfloat16)
                    rdmas[h][b].wait()
                    comm_ref[h + 1, b, :, :] = comm_ref[h + 1, b, :, :] + t
                    rdmas[h + 1][b].start()
                else:
                    t = chunk_f32(c, b)
                    rdmas[h][b].wait()
                    out_ref[b, :, :] = (
                        t + comm_ref[h + 1, b, :, :].astype(jnp.float32)
                    )

    return pl.pallas_call(
        body,
        out_shape=jax.ShapeDtypeStruct((B, s_per, N), jnp.float32),
        in_specs=[
            pl.BlockSpec(memory_space=pltpu.VMEM),
            pl.BlockSpec(memory_space=pltpu.VMEM),
        ],
        out_specs=pl.BlockSpec(memory_space=pltpu.VMEM),
        scratch_shapes=[
            pltpu.VMEM((N_Z, B, s_per, N), jnp.bfloat16),
            pltpu.VMEM((K, N), jnp.bfloat16),
            pltpu.SemaphoreType.DMA((N_Z - 1, B)),
            pltpu.SemaphoreType.DMA((N_Z - 1, B)),
        ],
        compiler_params=pltpu.CompilerParams(collective_id=0),
    )(x, Wo)
